# baseline (speedup 1.0000x reference)
import functools
import math

import jax
import jax.numpy as jnp
from jax import lax
from jax.experimental import pallas as pl
from jax.experimental.pallas import tpu as pltpu
from jax.experimental.pallas import tpu_sc as plsc

_SIZE = 32000
_PAD = 0
_SMOOTH = 0.1
_CONF = 1.0 - _SMOOTH
_EPS = _SMOOTH / (_SIZE - 2)
_C = _CONF * math.log(_CONF) + _SMOOTH * math.log(_EPS)

_ROWS_BLK = 128
_NW = 32
_BPW = 4096 // _NW
_GRAN = 16


def _stream_body(t_ref, x_ref, out_ref):
    i = pl.program_id(0)

    @pl.when(i == 0)
    def _init():
        out_ref[0, 0] = 0.0

    x = x_ref[...]
    t = t_ref[...]
    rowsum = jnp.sum(x, axis=1)
    z = x[:, 0]
    m = (t[:, 0] != _PAD).astype(jnp.float32)
    out_ref[0, 0] += jnp.sum(m * (_C - _EPS * (rowsum - z)))


def _combine_body(s1_ref, p_ref, out_ref):
    out_ref[0, 0] = s1_ref[0, 0] - jnp.sum(p_ref[...])


def _sc_gather_body(x_hbm, t_hbm, p_hbm, tv, gbuf, pv, sem0, sem1):
    sems = (sem0, sem1)
    wid = lax.axis_index("s") * 2 + lax.axis_index("c")
    base = wid * _BPW
    pltpu.sync_copy(t_hbm.at[pl.ds(base, _BPW)], tv)
    lanes = lax.iota(jnp.int32, 16)
    nb = _BPW // 32

    def fire(b):
        copies = []
        for c in range(2):
            cc = b * 2 + c
            t16 = tv[pl.ds(cc * 16, 16)]
            cs16 = lax.bitwise_and(t16, ~127)
            for k in range(16):
                r = cc * 16 + k
                cp = pltpu.make_async_copy(
                    x_hbm.at[
                        pl.ds(base + (r // 8) * 8, 8),
                        pl.ds(pl.multiple_of(cs16[k], 128), 128),
                    ],
                    gbuf.at[b % 2, c * 16 + k],
                    sems[b % 2],
                )
                cp.start()
                copies.append(cp)
        return copies

    def extract(b):
        for c in range(2):
            cc = b * 2 + c
            t16 = tv[pl.ds(cc * 16, 16)]
            g = plsc.load_gather(
                gbuf.at[b % 2],
                [c * 16 + lanes, lax.bitwise_and(lanes, 7), lax.bitwise_and(t16, 127)],
            )
            pv[pl.ds(cc * 16, 16)] = jnp.where(t16 != _PAD, (_CONF - _EPS) * g, 0.0)

    inflight = fire(0)
    for b in range(nb):
        nxt = fire(b + 1) if b + 1 < nb else []
        for cp in inflight:
            cp.wait()
        extract(b)
        inflight = nxt
    pltpu.sync_copy(pv, p_hbm.at[wid])


@jax.jit
def kernel(x, target):
    n = x.shape[0]
    t1 = target.astype(jnp.int32)
    t2 = t1.reshape(n, 1)

    p = pl.kernel(
        _sc_gather_body,
        mesh=plsc.VectorSubcoreMesh(core_axis_name="c", subcore_axis_name="s"),
        out_type=jax.ShapeDtypeStruct((_NW, _BPW), jnp.float32),
        scratch_types=[
            pltpu.VMEM((_BPW,), jnp.int32),
            pltpu.VMEM((2, 32, 8, 128), jnp.float32),
            pltpu.VMEM((_BPW,), jnp.float32),
            pltpu.SemaphoreType.DMA,
            pltpu.SemaphoreType.DMA,
        ],
        compiler_params=pltpu.CompilerParams(needs_layout_passes=False),
        cost_estimate=pl.CostEstimate(
            flops=2 * 4096, bytes_accessed=4096 * 4096, transcendentals=0
        ),
    )(x, t1)

    s1 = pl.pallas_call(
        _stream_body,
        grid=(n // _ROWS_BLK,),
        in_specs=[
            pl.BlockSpec((_ROWS_BLK, 1), lambda i: (i, 0)),
            pl.BlockSpec((_ROWS_BLK, _SIZE), lambda i: (i, 0)),
        ],
        out_specs=pl.BlockSpec(
            (1, 1), lambda i: (0, 0), memory_space=pltpu.SMEM
        ),
        out_shape=jax.ShapeDtypeStruct((1, 1), jnp.float32),
    )(t2, x)

    out = pl.pallas_call(
        _combine_body,
        out_specs=pl.BlockSpec(memory_space=pltpu.SMEM),
        out_shape=jax.ShapeDtypeStruct((1, 1), jnp.float32),
    )(s1, p)
    return out[0, 0]


# speedup vs baseline: 1.0130x; 1.0130x over previous
import math

import jax
import jax.numpy as jnp
from jax import lax
from jax.experimental import pallas as pl
from jax.experimental.pallas import tpu as pltpu
from jax.experimental.pallas import tpu_sc as plsc

_SIZE = 32000
_PAD = 0
_SMOOTH = 0.1
_CONF = 1.0 - _SMOOTH
_EPS = _SMOOTH / (_SIZE - 2)
_C = _CONF * math.log(_CONF) + _SMOOTH * math.log(_EPS)

_ROWS_BLK = 128
_NW = 32
_BPW = 4096 // _NW


def _stream_body(t_ref, x_ref, out_ref):
    i = pl.program_id(0)

    @pl.when(i == 0)
    def _init():
        out_ref[0, 0] = 0.0

    x = x_ref[...]
    t = t_ref[...]
    rowsum = jnp.sum(x, axis=1)
    z = x[:, 0]
    m = (t[:, 0] != _PAD).astype(jnp.float32)
    out_ref[0, 0] += jnp.sum(m * (_C - _EPS * (rowsum - z)))


def _combine_body(s1_ref, p_ref, out_ref):
    out_ref[0, 0] = s1_ref[0, 0] - jnp.sum(p_ref[...])


def _sc_gather_body(x_hbm, t_hbm, p_hbm, tv, gbuf, pv, sem0, sem1):
    sems = (sem0, sem1)
    wid = lax.axis_index("s") * 2 + lax.axis_index("c")
    base = wid * _BPW
    pltpu.sync_copy(t_hbm.at[pl.ds(base, _BPW)], tv)
    lanes = lax.iota(jnp.int32, 16)
    nb = _BPW // 32

    def fire(b):
        copies = []
        for c in range(2):
            cc = b * 2 + c
            t16 = tv[pl.ds(cc * 16, 16)]
            cs16 = lax.bitwise_and(t16, ~127)
            for k in range(16):
                r = cc * 16 + k
                cp = pltpu.make_async_copy(
                    x_hbm.at[
                        pl.ds(base + (r // 8) * 8, 8),
                        pl.ds(pl.multiple_of(cs16[k], 128), 128),
                    ],
                    gbuf.at[b % 2, c * 16 + k],
                    sems[b % 2],
                )
                cp.start()
                copies.append(cp)
        return copies

    def extract(b):
        for c in range(2):
            cc = b * 2 + c
            t16 = tv[pl.ds(cc * 16, 16)]
            g = plsc.load_gather(
                gbuf.at[b % 2],
                [c * 16 + lanes, lax.bitwise_and(lanes, 7), lax.bitwise_and(t16, 127)],
            )
            pv[pl.ds(cc * 16, 16)] = jnp.where(t16 != _PAD, (_CONF - _EPS) * g, 0.0)

    inflight = fire(0)
    for b in range(nb):
        nxt = fire(b + 1) if b + 1 < nb else []
        for cp in inflight:
            cp.wait()
        extract(b)
        inflight = nxt
    pltpu.sync_copy(pv, p_hbm.at[wid])


@jax.jit
def kernel(x, target):
    n = x.shape[0]
    t1 = target.astype(jnp.int32)
    t2 = t1.reshape(n, 1)

    p = pl.kernel(
        _sc_gather_body,
        mesh=plsc.VectorSubcoreMesh(core_axis_name="c", subcore_axis_name="s"),
        out_type=jax.ShapeDtypeStruct((_NW, _BPW), jnp.float32),
        scratch_types=[
            pltpu.VMEM((_BPW,), jnp.int32),
            pltpu.VMEM((2, 32, 8, 128), jnp.float32),
            pltpu.VMEM((_BPW,), jnp.float32),
            pltpu.SemaphoreType.DMA,
            pltpu.SemaphoreType.DMA,
        ],
        compiler_params=pltpu.CompilerParams(needs_layout_passes=False),
    )(x, t1)

    s1 = pl.pallas_call(
        _stream_body,
        grid=(n // _ROWS_BLK,),
        in_specs=[
            pl.BlockSpec((_ROWS_BLK, 1), lambda i: (i, 0)),
            pl.BlockSpec((_ROWS_BLK, _SIZE), lambda i: (i, 0)),
        ],
        out_specs=pl.BlockSpec(
            (1, 1), lambda i: (0, 0), memory_space=pltpu.SMEM
        ),
        out_shape=jax.ShapeDtypeStruct((1, 1), jnp.float32),
    )(t2, x)

    out = pl.pallas_call(
        _combine_body,
        out_specs=pl.BlockSpec(memory_space=pltpu.SMEM),
        out_shape=jax.ShapeDtypeStruct((1, 1), jnp.float32),
    )(s1, p)
    return out[0, 0]
